# baseline (speedup 1.0000x reference)
import jax
import jax.numpy as jnp
from jax import lax
from jax.experimental import pallas as pl
from jax.experimental.pallas import tpu as pltpu
from jax.experimental.pallas import tpu_sc as plsc

_B, _N, _M = 8, 4096, 256
_NCH = 256
_NB = _N // _NCH
_BIGF = 1e30
_BIGI = 1 << 30

_NW = 32
_SLOTS = _B * 2 * _M
_SPW = _SLOTS // _NW
_LANES = 16


def _tc_body(shape_ref, skelT_ref, out_cd, out_i1, out_i2,
             cda, m1, i1, m2, i2):
    b = pl.program_id(0)
    nb = pl.program_id(1)

    blk = shape_ref[0]
    px, py, pz = blk[:, 0:1], blk[:, 1:2], blk[:, 2:3]
    sk = skelT_ref[0]
    sx, sy, sz = sk[0:1, :], sk[1:2, :], sk[2:3, :]

    dxx = px - sx
    dyy = py - sy
    dzz = pz - sz
    d2m = dxx * dxx + dyy * dyy + dzz * dzz

    c1 = jnp.min(d2m, axis=1, keepdims=True)
    cd_part = jnp.sum(jnp.sqrt(c1 + 1e-12), keepdims=True).reshape(1, 1)

    ri = lax.broadcasted_iota(jnp.int32, (_NCH, _M), 0) + (b * _N + nb * _NCH)
    bm1 = jnp.min(d2m, axis=0, keepdims=True)
    bi1 = jnp.min(jnp.where(d2m == bm1, ri, _BIGI), axis=0, keepdims=True)
    sel1 = ri == bi1
    mk = jnp.where(sel1, _BIGF, d2m)
    bm2 = jnp.min(mk, axis=0, keepdims=True)
    bi2 = jnp.min(jnp.where(mk == bm2, ri, _BIGI), axis=0, keepdims=True)

    @pl.when(nb == 0)
    def _init():
        m1[...] = jnp.full((1, _M), _BIGF)
        m2[...] = jnp.full((1, _M), _BIGF)
        i1[...] = jnp.zeros((1, _M), jnp.int32)
        i2[...] = jnp.zeros((1, _M), jnp.int32)

    @pl.when((b == 0) & (nb == 0))
    def _init_acc():
        cda[...] = jnp.zeros((1, 1), jnp.float32)

    rm1, ri1, rm2, ri2 = m1[...], i1[...], m2[...], i2[...]
    c1lt = bm1 < rm1
    nm1 = jnp.where(c1lt, bm1, rm1)
    nv1 = jnp.where(c1lt, bi1, ri1)
    cm = jnp.where(c1lt, rm1, rm2)
    cv = jnp.where(c1lt, ri1, ri2)
    cbm = jnp.where(c1lt, bm2, bm1)
    cbv = jnp.where(c1lt, bi2, bi1)
    c2lt = cbm < cm
    nm2 = jnp.where(c2lt, cbm, cm)
    nv2 = jnp.where(c2lt, cbv, cv)
    m1[...] = nm1
    i1[...] = nv1
    m2[...] = nm2
    i2[...] = nv2

    cda[...] = cda[...] + cd_part

    @pl.when(nb == _NB - 1)
    def _fin_batch():
        cd2v = jnp.sum(jnp.sqrt(m1[...] + 1e-12), keepdims=True).reshape(1, 1)
        cda[...] = cda[...] + cd2v
        out_i1[0] = i1[...]
        out_i2[0] = i2[...]

    @pl.when((b == _B - 1) & (nb == _NB - 1))
    def _emit():
        out_cd[...] = cda[...]


def _tc_call(shape_xyz, skelT):
    return pl.pallas_call(
        _tc_body,
        grid=(_B, _NB),
        in_specs=[
            pl.BlockSpec((1, _NCH, 6), lambda b, nb: (b, nb, 0)),
            pl.BlockSpec((1, 3, _M), lambda b, nb: (b, 0, 0)),
        ],
        out_specs=[
            pl.BlockSpec((1, 1), lambda b, nb: (0, 0)),
            pl.BlockSpec((1, 1, _M), lambda b, nb: (b, 0, 0)),
            pl.BlockSpec((1, 1, _M), lambda b, nb: (b, 0, 0)),
        ],
        out_shape=[
            jax.ShapeDtypeStruct((1, 1), jnp.float32),
            jax.ShapeDtypeStruct((_B, 1, _M), jnp.int32),
            jax.ShapeDtypeStruct((_B, 1, _M), jnp.int32),
        ],
        scratch_shapes=[
            pltpu.VMEM((1, 1), jnp.float32),
            pltpu.VMEM((1, _M), jnp.float32),
            pltpu.VMEM((1, _M), jnp.int32),
            pltpu.VMEM((1, _M), jnp.float32),
            pltpu.VMEM((1, _M), jnp.int32),
        ],
    )(shape_xyz, skelT)


def _sc_body(normT_hbm, idx_hbm, nori_hbm, out_hbm,
             tbl_v, idx_v, nori_v, acc_v):
    cid = lax.axis_index("c")
    sid = lax.axis_index("s")
    wid = cid * 16 + sid
    pltpu.sync_copy(normT_hbm, tbl_v)
    pltpu.sync_copy(idx_hbm.at[wid], idx_v)
    pltpu.sync_copy(nori_hbm.at[wid], nori_v)
    acc = jnp.zeros((_LANES,), jnp.float32)
    for j in range(_SPW // _LANES):
        r = idx_v[pl.ds(j * _LANES, _LANES)]
        nx = plsc.load_gather(tbl_v, [r])
        ny = plsc.load_gather(tbl_v, [r + _B * _N])
        nz = plsc.load_gather(tbl_v, [r + 2 * _B * _N])
        ox = nori_v[0, pl.ds(j * _LANES, _LANES)]
        oy = nori_v[1, pl.ds(j * _LANES, _LANES)]
        oz = nori_v[2, pl.ds(j * _LANES, _LANES)]
        acc = acc + jnp.abs(nx * ox + ny * oy + nz * oz)
    acc_v[...] = acc
    pltpu.sync_copy(acc_v, out_hbm.at[wid])


def _sc_call(normT, idx_w, nori_w):
    return pl.kernel(
        _sc_body,
        out_type=jax.ShapeDtypeStruct((_NW, _LANES), jnp.float32),
        mesh=plsc.VectorSubcoreMesh(core_axis_name="c", subcore_axis_name="s"),
        compiler_params=pltpu.CompilerParams(needs_layout_passes=False),
        scratch_types=[
            pltpu.VMEM((3 * _B * _N,), jnp.float32),
            pltpu.VMEM((_SPW,), jnp.int32),
            pltpu.VMEM((3, _SPW), jnp.float32),
            pltpu.VMEM((_LANES,), jnp.float32),
        ],
    )(normT, idx_w, nori_w)


def kernel(shape_xyz, skel_xyz, skel_nori):
    skelT = jnp.transpose(skel_xyz, (0, 2, 1))
    cd_raw, idx1, idx2 = _tc_call(shape_xyz, skelT)

    idx_w = jnp.concatenate([idx1, idx2], axis=1)
    idx_w = idx_w.reshape(_NW, _SPW)
    noriT = jnp.transpose(skel_nori, (0, 2, 1))
    nori_s = jnp.stack([noriT, noriT], axis=1)
    nori_w = jnp.transpose(nori_s, (2, 0, 1, 3)).reshape(3, _SLOTS)
    nori_w = nori_w.reshape(3, _NW, _SPW).transpose(1, 0, 2)
    normT = shape_xyz[:, :, 3:6].reshape(_B * _N, 3).T.reshape(-1)

    parts = _sc_call(normT, idx_w, nori_w)
    return cd_raw[0, 0] * 1e-4 + 0.001 * (jnp.sum(parts) / (2.0 * _B))
